# baseline (speedup 1.0000x reference)
import functools
import math

import jax
import jax.numpy as jnp
from jax import lax
from jax.experimental import pallas as pl
from jax.experimental.pallas import tpu as pltpu

J = 5000
M = 10000
FOUT = 128
TM = 512
NT_M = math.ceil(M / TM)
NT_J = math.ceil(J / TM)
JPAD = NT_J * TM
NEG = -1e30

_HI = jax.lax.Precision.HIGHEST


def _sums_body(g_ref, rowsum_ref, colsum_ref, g8_ref):
    t = pl.program_id(0)
    g = g_ref[...]
    m_ids = t * TM + lax.broadcasted_iota(jnp.int32, (J, TM), 1)
    gm = jnp.where(m_ids < M, g, 0.0)

    @pl.when(t == 0)
    def _():
        rowsum_ref[...] = jnp.zeros_like(rowsum_ref)

    rowsum_ref[...] += jnp.sum(gm, axis=1, keepdims=True)
    colsum_ref[...] = jnp.sum(gm, axis=0, keepdims=True)

    @pl.when(t < NT_J)
    def _():
        pad = jnp.zeros((JPAD - J, TM), jnp.float32)
        g8_ref[...] = jnp.concatenate([g, pad], axis=0).astype(jnp.int8)


def _sums(graph):
    return pl.pallas_call(
        _sums_body,
        grid=(NT_M,),
        in_specs=[pl.BlockSpec((J, TM), lambda t: (0, t))],
        out_specs=[
            pl.BlockSpec((J, 1), lambda t: (0, 0)),
            pl.BlockSpec((1, TM), lambda t: (0, t)),
            pl.BlockSpec((JPAD, TM), lambda t: (0, jnp.minimum(t, NT_J - 1))),
        ],
        out_shape=[
            jax.ShapeDtypeStruct((J, 1), jnp.float32),
            jax.ShapeDtypeStruct((1, M), jnp.float32),
            jax.ShapeDtypeStruct((JPAD, JPAD), jnp.int8),
        ],
    )(graph)


def _hop_body(yt_ref, g_ref, out_ref):
    g = g_ref[...].astype(jnp.float32)
    out_ref[...] = jax.lax.dot_general(
        yt_ref[...], g, (((1,), (0,)), ((), ())),
        preferred_element_type=jnp.float32, precision=_HI)


def _hop(yt, g8):
    return pl.pallas_call(
        _hop_body,
        grid=(NT_J,),
        in_specs=[
            pl.BlockSpec((8, JPAD), lambda t: (0, 0)),
            pl.BlockSpec((JPAD, TM), lambda t: (0, t)),
        ],
        out_specs=pl.BlockSpec((8, TM), lambda t: (0, t)),
        out_shape=jax.ShapeDtypeStruct((8, JPAD), jnp.float32),
    )(yt, g8)


def _head_body(x_ref, w_ref, b_ref, out_ref):
    out_ref[...] = jax.lax.dot_general(
        x_ref[...], w_ref[...], (((1,), (0,)), ((), ())),
        preferred_element_type=jnp.float32, precision=_HI) + b_ref[...]


def _head(x18, w_tag, b_tag):
    return pl.pallas_call(
        _head_body,
        in_specs=[
            pl.BlockSpec((J, 24), lambda: (0, 0)),
            pl.BlockSpec((24, FOUT), lambda: (0, 0)),
            pl.BlockSpec((1, FOUT), lambda: (0, 0)),
        ],
        out_specs=pl.BlockSpec((J, FOUT), lambda: (0, 0)),
        out_shape=jax.ShapeDtypeStruct((J, FOUT), jnp.float32),
    )(x18, w_tag, b_tag)


def _masked_logit(t, v, a_col, a_row, prc_col):
    i_ids = lax.broadcasted_iota(jnp.int32, (J, TM), 0)
    m_ids = t * TM + lax.broadcasted_iota(jnp.int32, (J, TM), 1)
    s = a_col + a_row
    mask_left = jnp.logical_and(m_ids > i_ids, s == 0.0)
    p_left = jnp.where(mask_left, v, v - 10000.0)
    p_right = v - jnp.broadcast_to(prc_col, (J, TM))
    p = jnp.where(m_ids < J, p_left, p_right)
    return jnp.where(m_ids < M, p, NEG)


def _value_body(gt_ref, wt_ref, b_ref, a_col_ref, a_row_ref,
                prc_col_ref, val_ref, stats_ref, acc_ref):
    t = pl.program_id(0)
    v = jax.lax.dot_general(
        gt_ref[...], wt_ref[...], (((1,), (0,)), ((), ())),
        preferred_element_type=jnp.float32) + b_ref[...]
    val_ref[...] = v
    p = _masked_logit(t, v, a_col_ref[...], a_row_ref[...], prc_col_ref[...])
    tmax = jnp.max(p)
    tsum = jnp.sum(jnp.exp(p - tmax))
    acc_ref[0, t] = tmax
    acc_ref[1, t] = tsum

    @pl.when(t == NT_M - 1)
    def _():
        gmax = acc_ref[0, 0]
        for k in range(1, NT_M):
            gmax = jnp.maximum(gmax, acc_ref[0, k])
        gsum = jnp.float32(0.0)
        for k in range(NT_M):
            gsum = gsum + acc_ref[1, k] * jnp.exp(acc_ref[0, k] - gmax)
        stats_ref[0] = gmax
        stats_ref[1] = gsum


def _value(gt_j, w_lin_t, b_lin_row, a_col, a_row, prc_col):
    return pl.pallas_call(
        _value_body,
        grid=(NT_M,),
        in_specs=[
            pl.BlockSpec((J, FOUT), lambda t: (0, 0)),
            pl.BlockSpec((FOUT, TM), lambda t: (0, t)),
            pl.BlockSpec((1, TM), lambda t: (0, t)),
            pl.BlockSpec((J, 1), lambda t: (0, 0)),
            pl.BlockSpec((1, TM), lambda t: (0, t)),
            pl.BlockSpec((J, 1), lambda t: (0, 0)),
        ],
        out_specs=[
            pl.BlockSpec((J, TM), lambda t: (0, t)),
            pl.BlockSpec(memory_space=pltpu.SMEM),
        ],
        out_shape=[
            jax.ShapeDtypeStruct((J, M), jnp.float32),
            jax.ShapeDtypeStruct((2,), jnp.float32),
        ],
        scratch_shapes=[pltpu.SMEM((2, NT_M), jnp.float32)],
    )(gt_j, w_lin_t, b_lin_row, a_col, a_row, prc_col)


def _poss_body(gt_ref, wt_ref, b_ref, a_col_ref, a_row_ref,
               prc_col_ref, stats_ref, out_ref):
    t = pl.program_id(0)
    v = jax.lax.dot_general(
        gt_ref[...], wt_ref[...], (((1,), (0,)), ((), ())),
        preferred_element_type=jnp.float32) + b_ref[...]
    p = _masked_logit(t, v, a_col_ref[...], a_row_ref[...], prc_col_ref[...])
    gmax = stats_ref[0]
    inv = 1.0 / stats_ref[1]
    out_ref[...] = jnp.exp(p - gmax) * inv


def _poss(gt_j, w_lin_t, b_lin_row, a_col, a_row, prc_col, stats):
    return pl.pallas_call(
        _poss_body,
        grid=(NT_M,),
        in_specs=[
            pl.BlockSpec((J, FOUT), lambda t: (0, 0)),
            pl.BlockSpec((FOUT, TM), lambda t: (0, t)),
            pl.BlockSpec((1, TM), lambda t: (0, t)),
            pl.BlockSpec((J, 1), lambda t: (0, 0)),
            pl.BlockSpec((1, TM), lambda t: (0, t)),
            pl.BlockSpec((J, 1), lambda t: (0, 0)),
            pl.BlockSpec(memory_space=pltpu.SMEM),
        ],
        out_specs=pl.BlockSpec((J, TM), lambda t: (0, t)),
        out_shape=jax.ShapeDtypeStruct((J, M), jnp.float32),
    )(gt_j, w_lin_t, b_lin_row, a_col, a_row, prc_col, stats)


def kernel(h, L, W, P, N, Graph, W_tag, b_tag, W_lin, b_lin):
    f32 = jnp.float32
    order = jnp.argsort(-h, stable=True)
    h_s = h[order]
    L_s = L[order]

    rowsum, colsum, g8 = _sums(Graph)
    cs_l = colsum[0, :J]
    n = lax.rsqrt(jnp.maximum(cs_l, 1.0))

    def pad8(rows):
        r = jnp.stack(rows)
        return jnp.pad(r, ((0, 8 - len(rows)), (0, JPAD - J)))

    u0 = _hop(pad8([n * h_s, n * L_s, n]), g8)
    A1, B1, C1 = n * u0[0, :J], n * u0[1, :J], n * u0[2, :J]
    u1 = _hop(pad8([n * A1, n * B1, n * C1]), g8)
    A2, B2, C2 = n * u1[0, :J], n * u1[1, :J], n * u1[2, :J]

    one = jnp.ones((J,), f32)
    cols = [h_s, L_s, W * one, P * one, N * one, one,
            A1, B1, W * C1, P * C1, N * C1, C1,
            A2, B2, W * C2, P * C2, N * C2, C2]
    x18 = jnp.stack(cols, axis=1)
    x18 = jnp.pad(x18, ((0, 0), (0, 6)))
    w_tag24 = jnp.pad(W_tag, ((0, 6), (0, 0)))
    gt_j = _head(x18, w_tag24, b_tag.reshape(1, FOUT))

    w_lin_t = W_lin.T
    b_row = b_lin.reshape(1, M)
    a = rowsum[:, 0] + cs_l
    a_col = a.reshape(J, 1)
    a_row = jnp.concatenate(
        [a.reshape(1, J), jnp.ones((1, M - J), f32)], axis=1)
    prc_col = 10000.0 * rowsum

    value, stats = _value(gt_j, w_lin_t, b_row, a_col, a_row, prc_col)
    poss = _poss(gt_j, w_lin_t, b_row, a_col, a_row, prc_col, stats)
    return value, poss


# speedup vs baseline: 1.0113x; 1.0113x over previous
import functools
import math

import jax
import jax.numpy as jnp
from jax import lax
from jax.experimental import pallas as pl
from jax.experimental.pallas import tpu as pltpu

J = 5000
M = 10000
FOUT = 128
TM = 512
NT_M = math.ceil(M / TM)
NT_J = math.ceil(J / TM)
JPAD = NT_J * TM
NEG = -1e30

_HI = jax.lax.Precision.HIGHEST


def _sums_left_body(g_ref, rowsum_ref, colsum_ref, g8_ref):
    t = pl.program_id(0)
    g = g_ref[...]

    @pl.when(t == 0)
    def _():
        rowsum_ref[...] = jnp.zeros_like(rowsum_ref)

    rowsum_ref[...] += jnp.sum(g, axis=1, keepdims=True)
    colsum_ref[...] = jnp.sum(g, axis=0, keepdims=True)
    pad = jnp.zeros((JPAD - J, TM), jnp.float32)
    g8_ref[...] = jnp.concatenate([g, pad], axis=0).astype(jnp.int8)


def _sums_left(graph):
    return pl.pallas_call(
        _sums_left_body,
        grid=(NT_J,),
        in_specs=[pl.BlockSpec((J, TM), lambda t: (0, t))],
        out_specs=[
            pl.BlockSpec((J, 1), lambda t: (0, 0)),
            pl.BlockSpec((1, TM), lambda t: (0, t)),
            pl.BlockSpec((JPAD, TM), lambda t: (0, t)),
        ],
        out_shape=[
            jax.ShapeDtypeStruct((J, 1), jnp.float32),
            jax.ShapeDtypeStruct((1, JPAD), jnp.float32),
            jax.ShapeDtypeStruct((JPAD, JPAD), jnp.int8),
        ],
    )(graph)


def _sums_right_body(g_ref, rowsum_ref):
    t = pl.program_id(0)
    g = g_ref[...]
    m_ids = (NT_J + t) * TM + lax.broadcasted_iota(jnp.int32, (J, TM), 1)
    g = jnp.where(m_ids < M, g, 0.0)

    @pl.when(t == 0)
    def _():
        rowsum_ref[...] = jnp.zeros_like(rowsum_ref)

    rowsum_ref[...] += jnp.sum(g, axis=1, keepdims=True)


def _sums_right(graph):
    return pl.pallas_call(
        _sums_right_body,
        grid=(NT_M - NT_J,),
        in_specs=[pl.BlockSpec((J, TM), lambda t: (0, NT_J + t))],
        out_specs=pl.BlockSpec((J, 1), lambda t: (0, 0)),
        out_shape=jax.ShapeDtypeStruct((J, 1), jnp.float32),
    )(graph)


def _hop_body(yt_ref, g_ref, out_ref):
    g = g_ref[...].astype(jnp.float32)
    out_ref[...] = jax.lax.dot_general(
        yt_ref[...], g, (((1,), (0,)), ((), ())),
        preferred_element_type=jnp.float32, precision=_HI)


def _hop(yt, g8):
    return pl.pallas_call(
        _hop_body,
        grid=(NT_J,),
        in_specs=[
            pl.BlockSpec((8, JPAD), lambda t: (0, 0)),
            pl.BlockSpec((JPAD, TM), lambda t: (0, t)),
        ],
        out_specs=pl.BlockSpec((8, TM), lambda t: (0, t)),
        out_shape=jax.ShapeDtypeStruct((8, JPAD), jnp.float32),
    )(yt, g8)


def _head_body(x_ref, w_ref, b_ref, out_ref):
    out_ref[...] = jax.lax.dot_general(
        x_ref[...], w_ref[...], (((1,), (0,)), ((), ())),
        preferred_element_type=jnp.float32, precision=_HI) + b_ref[...]


def _head(x18, w_tag, b_tag):
    return pl.pallas_call(
        _head_body,
        in_specs=[
            pl.BlockSpec((J, 24), lambda: (0, 0)),
            pl.BlockSpec((24, FOUT), lambda: (0, 0)),
            pl.BlockSpec((1, FOUT), lambda: (0, 0)),
        ],
        out_specs=pl.BlockSpec((J, FOUT), lambda: (0, 0)),
        out_shape=jax.ShapeDtypeStruct((J, FOUT), jnp.float32),
    )(x18, w_tag, b_tag)


def _masked_logit(t, v, a_col, a_row, prc_col):
    i_ids = lax.broadcasted_iota(jnp.int32, (J, TM), 0)
    m_ids = t * TM + lax.broadcasted_iota(jnp.int32, (J, TM), 1)
    s = a_col + a_row
    mask_left = jnp.logical_and(m_ids > i_ids, s == 0.0)
    p_left = jnp.where(mask_left, v, v - 10000.0)
    p_right = v - jnp.broadcast_to(prc_col, (J, TM))
    p = jnp.where(m_ids < J, p_left, p_right)
    return jnp.where(m_ids < M, p, NEG)


def _value_body(gt_ref, wt_ref, b_ref, a_col_ref, a_row_ref,
                prc_col_ref, val_ref, stats_ref, acc_ref):
    t = pl.program_id(0)
    v = jax.lax.dot_general(
        gt_ref[...], wt_ref[...], (((1,), (0,)), ((), ())),
        preferred_element_type=jnp.float32) + b_ref[...]
    val_ref[...] = v
    p = _masked_logit(t, v, a_col_ref[...], a_row_ref[...], prc_col_ref[...])
    tmax = jnp.max(p)
    tsum = jnp.sum(jnp.exp(p - tmax))
    acc_ref[0, t] = tmax
    acc_ref[1, t] = tsum

    @pl.when(t == NT_M - 1)
    def _():
        gmax = acc_ref[0, 0]
        for k in range(1, NT_M):
            gmax = jnp.maximum(gmax, acc_ref[0, k])
        gsum = jnp.float32(0.0)
        for k in range(NT_M):
            gsum = gsum + acc_ref[1, k] * jnp.exp(acc_ref[0, k] - gmax)
        stats_ref[0] = gmax
        stats_ref[1] = gsum


def _value(gt_j, w_lin_t, b_lin_row, a_col, a_row, prc_col):
    return pl.pallas_call(
        _value_body,
        grid=(NT_M,),
        in_specs=[
            pl.BlockSpec((J, FOUT), lambda t: (0, 0)),
            pl.BlockSpec((FOUT, TM), lambda t: (0, t)),
            pl.BlockSpec((1, TM), lambda t: (0, t)),
            pl.BlockSpec((J, 1), lambda t: (0, 0)),
            pl.BlockSpec((1, TM), lambda t: (0, t)),
            pl.BlockSpec((J, 1), lambda t: (0, 0)),
        ],
        out_specs=[
            pl.BlockSpec((J, TM), lambda t: (0, t)),
            pl.BlockSpec(memory_space=pltpu.SMEM),
        ],
        out_shape=[
            jax.ShapeDtypeStruct((J, M), jnp.float32),
            jax.ShapeDtypeStruct((2,), jnp.float32),
        ],
        scratch_shapes=[pltpu.SMEM((2, NT_M), jnp.float32)],
    )(gt_j, w_lin_t, b_lin_row, a_col, a_row, prc_col)


def _poss_body(gt_ref, wt_ref, b_ref, a_col_ref, a_row_ref,
               prc_col_ref, stats_ref, out_ref):
    t = pl.program_id(0)
    v = jax.lax.dot_general(
        gt_ref[...], wt_ref[...], (((1,), (0,)), ((), ())),
        preferred_element_type=jnp.float32) + b_ref[...]
    p = _masked_logit(t, v, a_col_ref[...], a_row_ref[...], prc_col_ref[...])
    gmax = stats_ref[0]
    inv = 1.0 / stats_ref[1]
    out_ref[...] = jnp.exp(p - gmax) * inv


def _poss(gt_j, w_lin_t, b_lin_row, a_col, a_row, prc_col, stats):
    return pl.pallas_call(
        _poss_body,
        grid=(NT_M,),
        in_specs=[
            pl.BlockSpec((J, FOUT), lambda t: (0, 0)),
            pl.BlockSpec((FOUT, TM), lambda t: (0, t)),
            pl.BlockSpec((1, TM), lambda t: (0, t)),
            pl.BlockSpec((J, 1), lambda t: (0, 0)),
            pl.BlockSpec((1, TM), lambda t: (0, t)),
            pl.BlockSpec((J, 1), lambda t: (0, 0)),
            pl.BlockSpec(memory_space=pltpu.SMEM),
        ],
        out_specs=pl.BlockSpec((J, TM), lambda t: (0, t)),
        out_shape=jax.ShapeDtypeStruct((J, M), jnp.float32),
    )(gt_j, w_lin_t, b_lin_row, a_col, a_row, prc_col, stats)


def kernel(h, L, W, P, N, Graph, W_tag, b_tag, W_lin, b_lin):
    f32 = jnp.float32
    order = jnp.argsort(-h, stable=True)
    h_s = h[order]
    L_s = L[order]

    rowsum_l, colsum, g8 = _sums_left(Graph)
    rowsum = rowsum_l + _sums_right(Graph)
    cs_l = colsum[0, :J]
    n = lax.rsqrt(jnp.maximum(cs_l, 1.0))

    def pad8(rows):
        r = jnp.stack(rows)
        return jnp.pad(r, ((0, 8 - len(rows)), (0, JPAD - J)))

    u0 = _hop(pad8([n * h_s, n * L_s, n]), g8)
    A1, B1, C1 = n * u0[0, :J], n * u0[1, :J], n * u0[2, :J]
    u1 = _hop(pad8([n * A1, n * B1, n * C1]), g8)
    A2, B2, C2 = n * u1[0, :J], n * u1[1, :J], n * u1[2, :J]

    one = jnp.ones((J,), f32)
    cols = [h_s, L_s, W * one, P * one, N * one, one,
            A1, B1, W * C1, P * C1, N * C1, C1,
            A2, B2, W * C2, P * C2, N * C2, C2]
    x18 = jnp.stack(cols, axis=1)
    x18 = jnp.pad(x18, ((0, 0), (0, 6)))
    w_tag24 = jnp.pad(W_tag, ((0, 6), (0, 0)))
    gt_j = _head(x18, w_tag24, b_tag.reshape(1, FOUT))

    w_lin_t = W_lin.T
    b_row = b_lin.reshape(1, M)
    a = rowsum[:, 0] + cs_l
    a_col = a.reshape(J, 1)
    a_row = jnp.concatenate(
        [a.reshape(1, J), jnp.ones((1, M - J), f32)], axis=1)
    prc_col = 10000.0 * rowsum

    value, stats = _value(gt_j, w_lin_t, b_row, a_col, a_row, prc_col)
    poss = _poss(gt_j, w_lin_t, b_row, a_col, a_row, prc_col, stats)
    return value, poss


# speedup vs baseline: 1.2687x; 1.2546x over previous
import functools
import math

import jax
import jax.numpy as jnp
from jax import lax
from jax.experimental import pallas as pl
from jax.experimental.pallas import tpu as pltpu

J = 5000
M = 10000
FOUT = 128
TM = 512
NT_M = math.ceil(M / TM)
NT_J = math.ceil(J / TM)
JPAD = NT_J * TM
NEG = -1e30

_HI = jax.lax.Precision.HIGHEST


def _sums_left_body(g_ref, rowsum_ref, colsum_ref, g8_ref):
    t = pl.program_id(0)
    g = g_ref[...]

    @pl.when(t == 0)
    def _():
        rowsum_ref[...] = jnp.zeros_like(rowsum_ref)

    rowsum_ref[...] += jnp.sum(g, axis=1, keepdims=True)
    colsum_ref[...] = jnp.sum(g, axis=0, keepdims=True)
    pad = jnp.zeros((JPAD - J, TM), jnp.float32)
    g8_ref[...] = jnp.concatenate([g, pad], axis=0).astype(jnp.int8)


def _sums_left(graph):
    return pl.pallas_call(
        _sums_left_body,
        grid=(NT_J,),
        in_specs=[pl.BlockSpec((J, TM), lambda t: (0, t))],
        out_specs=[
            pl.BlockSpec((J, 1), lambda t: (0, 0)),
            pl.BlockSpec((1, TM), lambda t: (0, t)),
            pl.BlockSpec((JPAD, TM), lambda t: (0, t)),
        ],
        out_shape=[
            jax.ShapeDtypeStruct((J, 1), jnp.float32),
            jax.ShapeDtypeStruct((1, JPAD), jnp.float32),
            jax.ShapeDtypeStruct((JPAD, JPAD), jnp.int8),
        ],
    )(graph)


def _sums_right_body(g_ref, rowsum_ref):
    t = pl.program_id(0)
    g = g_ref[...]
    m_ids = (NT_J + t) * TM + lax.broadcasted_iota(jnp.int32, (J, TM), 1)
    g = jnp.where(m_ids < M, g, 0.0)

    @pl.when(t == 0)
    def _():
        rowsum_ref[...] = jnp.zeros_like(rowsum_ref)

    rowsum_ref[...] += jnp.sum(g, axis=1, keepdims=True)


def _sums_right(graph):
    return pl.pallas_call(
        _sums_right_body,
        grid=(NT_M - NT_J,),
        in_specs=[pl.BlockSpec((J, TM), lambda t: (0, NT_J + t))],
        out_specs=pl.BlockSpec((J, 1), lambda t: (0, 0)),
        out_shape=jax.ShapeDtypeStruct((J, 1), jnp.float32),
    )(graph)


def _hop_body(yt_ref, g_ref, out_ref):
    g = g_ref[...].astype(jnp.float32)
    out_ref[...] = jax.lax.dot_general(
        yt_ref[...], g, (((1,), (0,)), ((), ())),
        preferred_element_type=jnp.float32)


def _hop(yt, g8):
    return pl.pallas_call(
        _hop_body,
        grid=(NT_J,),
        in_specs=[
            pl.BlockSpec((8, JPAD), lambda t: (0, 0)),
            pl.BlockSpec((JPAD, TM), lambda t: (0, t)),
        ],
        out_specs=pl.BlockSpec((8, TM), lambda t: (0, t)),
        out_shape=jax.ShapeDtypeStruct((8, JPAD), jnp.float32),
    )(yt, g8)


def _head_body(x_ref, w_ref, b_ref, out_ref):
    out_ref[...] = jax.lax.dot_general(
        x_ref[...], w_ref[...], (((1,), (0,)), ((), ())),
        preferred_element_type=jnp.float32, precision=_HI) + b_ref[...]


def _head(x18, w_tag, b_tag):
    return pl.pallas_call(
        _head_body,
        in_specs=[
            pl.BlockSpec((J, 24), lambda: (0, 0)),
            pl.BlockSpec((24, FOUT), lambda: (0, 0)),
            pl.BlockSpec((1, FOUT), lambda: (0, 0)),
        ],
        out_specs=pl.BlockSpec((J, FOUT), lambda: (0, 0)),
        out_shape=jax.ShapeDtypeStruct((J, FOUT), jnp.float32),
    )(x18, w_tag, b_tag)


def _masked_logit(v, rv, cv):
    pen = cv[:, 1:2] + rv[2:3] * cv[:, 2:3]
    return jnp.where(rv[1:2] > cv[:, 0:1], v, v - pen) + rv[3:4]


def _value_body(gt_ref, wt_ref, rv_ref, cv_ref, val_ref, stats_ref, acc_ref):
    t = pl.program_id(0)
    rv = rv_ref[...]
    v = jax.lax.dot_general(
        gt_ref[...], wt_ref[...], (((1,), (0,)), ((), ())),
        preferred_element_type=jnp.float32) + rv[0:1]
    val_ref[...] = v
    p = _masked_logit(v, rv, cv_ref[...])
    tmax = jnp.max(p)
    tsum = jnp.sum(jnp.exp(p - tmax))
    acc_ref[0, t] = tmax
    acc_ref[1, t] = tsum

    @pl.when(t == NT_M - 1)
    def _():
        gmax = acc_ref[0, 0]
        for k in range(1, NT_M):
            gmax = jnp.maximum(gmax, acc_ref[0, k])
        gsum = jnp.float32(0.0)
        for k in range(NT_M):
            gsum = gsum + acc_ref[1, k] * jnp.exp(acc_ref[0, k] - gmax)
        stats_ref[0] = gmax
        stats_ref[1] = gsum


def _value(gt_j, wt_pad, rv, cv):
    return pl.pallas_call(
        _value_body,
        grid=(NT_M,),
        in_specs=[
            pl.BlockSpec((J, FOUT), lambda t: (0, 0)),
            pl.BlockSpec((FOUT, TM), lambda t: (0, t)),
            pl.BlockSpec((4, TM), lambda t: (0, t)),
            pl.BlockSpec((J, 3), lambda t: (0, 0)),
        ],
        out_specs=[
            pl.BlockSpec((J, TM), lambda t: (0, t)),
            pl.BlockSpec(memory_space=pltpu.SMEM),
        ],
        out_shape=[
            jax.ShapeDtypeStruct((J, M), jnp.float32),
            jax.ShapeDtypeStruct((2,), jnp.float32),
        ],
        scratch_shapes=[pltpu.SMEM((2, NT_M), jnp.float32)],
    )(gt_j, wt_pad, rv, cv)


def _poss_body(gt_ref, wt_ref, rv_ref, cv_ref, stats_ref, out_ref):
    rv = rv_ref[...]
    v = jax.lax.dot_general(
        gt_ref[...], wt_ref[...], (((1,), (0,)), ((), ())),
        preferred_element_type=jnp.float32) + rv[0:1]
    p = _masked_logit(v, rv, cv_ref[...])
    gmax = stats_ref[0]
    inv = 1.0 / stats_ref[1]
    out_ref[...] = jnp.exp(p - gmax) * inv


def _poss(gt_j, wt_pad, rv, cv, stats):
    return pl.pallas_call(
        _poss_body,
        grid=(NT_M,),
        in_specs=[
            pl.BlockSpec((J, FOUT), lambda t: (0, 0)),
            pl.BlockSpec((FOUT, TM), lambda t: (0, t)),
            pl.BlockSpec((4, TM), lambda t: (0, t)),
            pl.BlockSpec((J, 3), lambda t: (0, 0)),
            pl.BlockSpec(memory_space=pltpu.SMEM),
        ],
        out_specs=pl.BlockSpec((J, TM), lambda t: (0, t)),
        out_shape=jax.ShapeDtypeStruct((J, M), jnp.float32),
    )(gt_j, wt_pad, rv, cv, stats)


def kernel(h, L, W, P, N, Graph, W_tag, b_tag, W_lin, b_lin):
    f32 = jnp.float32
    order = jnp.argsort(-h, stable=True)
    h_s = h[order]
    L_s = L[order]

    rowsum_l, colsum, g8 = _sums_left(Graph)
    rowsum = rowsum_l + _sums_right(Graph)
    cs_l = colsum[0, :J]
    n = lax.rsqrt(jnp.maximum(cs_l, 1.0))

    def pad8(rows):
        r = jnp.stack(rows)
        return jnp.pad(r, ((0, 8 - len(rows)), (0, JPAD - J)))

    u0 = _hop(pad8([n * h_s, n * L_s, n]), g8)
    A1, B1, C1 = n * u0[0, :J], n * u0[1, :J], n * u0[2, :J]
    u1 = _hop(pad8([n * A1, n * B1, n * C1]), g8)
    A2, B2, C2 = n * u1[0, :J], n * u1[1, :J], n * u1[2, :J]

    one = jnp.ones((J,), f32)
    cols = [h_s, L_s, W * one, P * one, N * one, one,
            A1, B1, W * C1, P * C1, N * C1, C1,
            A2, B2, W * C2, P * C2, N * C2, C2]
    x18 = jnp.stack(cols, axis=1)
    x18 = jnp.pad(x18, ((0, 0), (0, 6)))
    w_tag24 = jnp.pad(W_tag, ((0, 6), (0, 0)))
    gt_j = _head(x18, w_tag24, b_tag.reshape(1, FOUT))

    MP = NT_M * TM
    wt_pad = jnp.pad(W_lin.T, ((0, 0), (0, MP - M)))
    a = rowsum[:, 0] + cs_l
    idx = jnp.arange(J, dtype=f32)
    c_row = jnp.concatenate(
        [jnp.where(a == 0.0, idx, -1e9), jnp.full((MP - J,), -1e9, f32)])
    jl_row = jnp.concatenate([jnp.ones((J,), f32), jnp.zeros((MP - J,), f32)])
    z_row = jnp.concatenate([jnp.zeros((M,), f32), jnp.full((MP - M,), NEG, f32)])
    b_row = jnp.concatenate([b_lin, jnp.zeros((MP - M,), f32)])
    rv = jnp.stack([b_row, c_row, jl_row, z_row])
    c_col = jnp.where(a == 0.0, idx, 2e9)
    prc = 10000.0 * rowsum[:, 0]
    cv = jnp.stack([c_col, prc, 10000.0 - prc], axis=1)

    value, stats = _value(gt_j, wt_pad, rv, cv)
    poss = _poss(gt_j, wt_pad, rv, cv, stats)
    return value, poss
